# baseline (speedup 1.0000x reference)
import functools

import jax
import jax.numpy as jnp
from jax import lax
from jax.experimental import pallas as pl
from jax.experimental.pallas import tpu as pltpu
from jax.experimental.pallas import tpu_sc as plsc

D_MODEL = 128
BATCH = 4096
HIST = 50

NC = 2
NS = 16
NW = NC * NS

B_PER_TILE = BATCH // NW
GROUP = 4
NGROUP = B_PER_TILE // GROUP
TABLE_ROWS = 367

_mesh = plsc.VectorSubcoreMesh(core_axis_name="c", subcore_axis_name="s")


@functools.partial(
    pl.kernel,
    mesh=_mesh,
    out_type=jax.ShapeDtypeStruct((BATCH, HIST, D_MODEL), jnp.float32),
    scratch_types=[
        pltpu.VMEM((B_PER_TILE, HIST), jnp.int32),
        pltpu.VMEM((2, GROUP, HIST, D_MODEL), jnp.float32),
        pltpu.VMEM_SHARED((TABLE_ROWS, D_MODEL), jnp.float32),
        pltpu.SemaphoreType.DMA,
        pltpu.SemaphoreType.DMA,
        pltpu.SemaphoreType.DMA,
        pltpu.SemaphoreType.DMA,
    ],
)
def _pe_gather(table_hbm, idx_hbm, out_hbm, idx_v, rows_v, table_v, sem_a,
               sem_b, sem_st0, sem_st1):
    wid = lax.axis_index("s") * NC + lax.axis_index("c")
    base = wid * B_PER_TILE
    pltpu.sync_copy(idx_hbm.at[pl.ds(base, B_PER_TILE)], idx_v)
    sid = lax.axis_index("s")

    @pl.when(sid < NS - 1)
    def _():
        pltpu.sync_copy(table_hbm.at[pl.ds(sid * 24, 24)],
                        table_v.at[pl.ds(sid * 24, 24)])

    @pl.when(sid == NS - 1)
    def _():
        pltpu.sync_copy(table_hbm.at[pl.ds((NS - 1) * 24, TABLE_ROWS - (NS - 1) * 24)],
                        table_v.at[pl.ds((NS - 1) * 24, TABLE_ROWS - (NS - 1) * 24)])

    plsc.subcore_barrier()

    def gather_group(g, buf, sem):
        for r in range(GROUP):
            pltpu.async_copy(
                table_v.at[idx_v.at[g * GROUP + r]], rows_v.at[buf, r], sem)

    def drain_group(g, buf, sem):
        for r in range(GROUP):
            pltpu.make_async_copy(
                table_v.at[idx_v.at[g * GROUP + r]], rows_v.at[buf, r],
                sem).wait()

    def store_group(g, buf, sem):
        pltpu.async_copy(
            rows_v.at[buf], out_hbm.at[pl.ds(base + g * GROUP, GROUP)], sem)

    def store_wait(buf, sem):
        pltpu.make_async_copy(
            rows_v.at[buf], out_hbm.at[pl.ds(base, GROUP)], sem).wait()

    gather_group(0, 0, sem_a)

    def pair_body(i, carry):
        g = 2 * i

        @pl.when(i > 0)
        def _():
            store_wait(1, sem_st1)

        gather_group(g + 1, 1, sem_b)
        drain_group(g, 0, sem_a)
        store_group(g, 0, sem_st0)

        @pl.when(g + 2 < NGROUP)
        def _():
            store_wait(0, sem_st0)
            gather_group(g + 2, 0, sem_a)

        drain_group(g + 1, 1, sem_b)
        store_group(g + 1, 1, sem_st1)
        return carry

    lax.fori_loop(0, NGROUP // 2, pair_body, 0)
    store_wait(0, sem_st0)
    store_wait(1, sem_st1)


def kernel(doy, pe):
    return _pe_gather(pe, doy)


# speedup vs baseline: 1.0073x; 1.0073x over previous
import functools

import jax
import jax.numpy as jnp
from jax import lax
from jax.experimental import pallas as pl
from jax.experimental.pallas import tpu as pltpu
from jax.experimental.pallas import tpu_sc as plsc

D_MODEL = 128
BATCH = 4096
HIST = 50

NC = 2
NS = 16
NW = NC * NS

B_PER_TILE = BATCH // NW
GROUP = 2
NBUF = 4
NGROUP = B_PER_TILE // GROUP
TABLE_ROWS = 367

_mesh = plsc.VectorSubcoreMesh(core_axis_name="c", subcore_axis_name="s")


@functools.partial(
    pl.kernel,
    mesh=_mesh,
    out_type=jax.ShapeDtypeStruct((BATCH, HIST, D_MODEL), jnp.float32),
    scratch_types=[
        pltpu.VMEM((B_PER_TILE, HIST), jnp.int32),
        pltpu.VMEM((NBUF, GROUP, HIST, D_MODEL), jnp.float32),
        pltpu.VMEM_SHARED((TABLE_ROWS, D_MODEL), jnp.float32),
        pltpu.SemaphoreType.DMA((NBUF,)),
        pltpu.SemaphoreType.DMA((NBUF,)),
    ],
)
def _pe_gather(table_hbm, idx_hbm, out_hbm, idx_v, rows_v, table_v, sem_g,
               sem_s):
    wid = lax.axis_index("s") * NC + lax.axis_index("c")
    base = wid * B_PER_TILE
    @pl.when(lax.axis_index("s") == 0)
    def _():
        pltpu.sync_copy(table_hbm, table_v)

    plsc.subcore_barrier()
    pltpu.sync_copy(idx_hbm.at[pl.ds(base, B_PER_TILE)], idx_v)

    def gather_group(g, buf):
        for r in range(GROUP):
            pltpu.async_copy(
                table_v.at[idx_v.at[g * GROUP + r]], rows_v.at[buf, r],
                sem_g.at[buf])

    def drain_group(g, buf):
        for r in range(GROUP):
            pltpu.make_async_copy(
                table_v.at[idx_v.at[g * GROUP + r]], rows_v.at[buf, r],
                sem_g.at[buf]).wait()

    def store_group(g, buf):
        pltpu.async_copy(
            rows_v.at[buf], out_hbm.at[pl.ds(base + g * GROUP, GROUP)],
            sem_s.at[buf])

    def store_wait(buf):
        pltpu.make_async_copy(
            rows_v.at[buf], out_hbm.at[pl.ds(base, GROUP)],
            sem_s.at[buf]).wait()

    for b in range(NBUF):
        gather_group(b, b)

    def body(i, carry):
        g0 = i * NBUF
        for b in range(NBUF):
            drain_group(g0 + b, b)
            store_group(g0 + b, b)
        for b in range(NBUF):
            @pl.when(g0 + NBUF + b < NGROUP)
            def _():
                store_wait(b)
                gather_group(g0 + NBUF + b, b)

        return carry

    lax.fori_loop(0, NGROUP // NBUF, body, 0)
    for b in range(NBUF):
        store_wait(b)


def kernel(doy, pe):
    return _pe_gather(pe, doy)
